# baseline (speedup 1.0000x reference)
import functools

import jax
import jax.numpy as jnp
from jax import lax
from jax.experimental import pallas as pl
from jax.experimental.pallas import tpu as pltpu
from jax.experimental.pallas import tpu_sc as plsc

N = 10000
NPAD = 10240
E = 320000
NC = 2
NS = 16
NW = NC * NS
CH = 128
CPW = 80
EPAD = NW * CPW * CH
ROWS_PT = NPAD // NS
PAD_NODE = N
NBUF = 8
LOOKAHEAD = 4

CPW_AGG = (152, 8)
CPW_DEG = (104, 56)

_MESH = plsc.VectorSubcoreMesh(
    core_axis_name="c", subcore_axis_name="s", num_cores=NC, num_subcores=NS
)


def _make_deg_kernel():
    @functools.partial(
        pl.kernel,
        out_type=jax.ShapeDtypeStruct((NC, NPAD, 16), jnp.float32),
        mesh=_MESH,
        scratch_types=[
            pltpu.VMEM((CPW_DEG[0], CH), jnp.int32),
            pltpu.VMEM((CH, 16), jnp.float32),
            pltpu.VMEM_SHARED((NPAD, 16), jnp.float32),
            pltpu.SemaphoreType.DMA,
        ],
        compiler_params=pltpu.CompilerParams(use_tc_tiling_on_sc=False),
    )
    def deg_kernel(didx_hbm, zeros_hbm, ones_hbm, out_hbm, didx_v, ones_v, acc,
                   sem):
        c = lax.axis_index("c")
        s = lax.axis_index("s")
        r0 = s * ROWS_PT
        pltpu.sync_copy(zeros_hbm.at[pl.ds(r0, ROWS_PT)], acc.at[pl.ds(r0, ROWS_PT)])
        pltpu.sync_copy(ones_hbm, ones_v)

        @pl.when(c == 0)
        def _load0():
            pltpu.sync_copy(didx_hbm.at[pl.ds(s * CPW_DEG[0], CPW_DEG[0])], didx_v)

        @pl.when(c == 1)
        def _load1():
            pltpu.sync_copy(
                didx_hbm.at[pl.ds(NS * CPW_DEG[0] + s * CPW_DEG[1], CPW_DEG[1])],
                didx_v.at[pl.ds(0, CPW_DEG[1])])

        plsc.subcore_barrier()
        cpw = jnp.where(c == 0, CPW_DEG[0], CPW_DEG[1])

        def body(j, carry):
            pltpu.async_copy(ones_v, acc.at[didx_v.at[j]], sem, add=True)
            return carry

        lax.fori_loop(0, cpw, body, 0)

        def drain(j, carry):
            pltpu.make_async_copy(ones_v, acc.at[didx_v.at[j]], sem).wait()
            return carry

        lax.fori_loop(0, cpw, drain, 0)
        plsc.subcore_barrier()
        pltpu.sync_copy(acc.at[pl.ds(r0, ROWS_PT)], out_hbm.at[c, pl.ds(r0, ROWS_PT)])

    return deg_kernel


def _make_agg_kernel(D):
    @functools.partial(
        pl.kernel,
        out_type=jax.ShapeDtypeStruct((NC, NPAD, D), jnp.float32),
        mesh=_MESH,
        scratch_types=[
            pltpu.VMEM((CPW_AGG[0], CH), jnp.int32),
            pltpu.VMEM((NBUF, CH), jnp.int32),
            pltpu.VMEM((NBUF, CH, D), jnp.float32),
            pltpu.VMEM_SHARED((NPAD, D), jnp.float32),
            pltpu.SemaphoreType.DMA((NBUF,)),
            pltpu.SemaphoreType.DMA((NBUF,)),
            pltpu.SemaphoreType.DMA((NBUF,)),
        ],
        compiler_params=pltpu.CompilerParams(use_tc_tiling_on_sc=False),
    )
    def agg_kernel(hp_hbm, sidx_hbm, didx_hbm, zeros_hbm, out_hbm,
                   sidx_v, didx_ring, rows_v, acc, gsem, ssem, dsem):
        c = lax.axis_index("c")
        s = lax.axis_index("s")
        r0 = s * ROWS_PT
        with jax.named_scope("zinit"):
            pltpu.sync_copy(zeros_hbm.at[pl.ds(r0, ROWS_PT)], acc.at[pl.ds(r0, ROWS_PT)])

        with jax.named_scope("idxload"):
            @pl.when(c == 0)
            def _load0():
                pltpu.sync_copy(sidx_hbm.at[pl.ds(s * CPW_AGG[0], CPW_AGG[0])], sidx_v)

            @pl.when(c == 1)
            def _load1():
                base = NS * CPW_AGG[0] + s * CPW_AGG[1]
                pltpu.sync_copy(sidx_hbm.at[pl.ds(base, CPW_AGG[1])],
                                sidx_v.at[pl.ds(0, CPW_AGG[1])])

        with jax.named_scope("barrier1"):
            plsc.subcore_barrier()

        def run(cpw, rowbase):
            for b in range(LOOKAHEAD):
                pltpu.async_copy(didx_hbm.at[rowbase + b], didx_ring.at[b],
                                 dsem.at[b])
                pltpu.async_copy(hp_hbm.at[sidx_v.at[b]], rows_v.at[b], gsem.at[b])

            def step_body(step, carry):
                base = step * NBUF
                for b in range(NBUF):
                    j = base + b
                    pltpu.make_async_copy(
                        hp_hbm.at[sidx_v.at[j]], rows_v.at[b], gsem.at[b]).wait()
                    pltpu.make_async_copy(
                        didx_hbm.at[rowbase + j], didx_ring.at[b],
                        dsem.at[b]).wait()
                    pltpu.async_copy(
                        rows_v.at[b], acc.at[didx_ring.at[b]], ssem.at[b],
                        add=True)
                    bn = (b + LOOKAHEAD) % NBUF
                    jg = j + LOOKAHEAD
                    jp = jg - NBUF

                    @pl.when(jp >= 0)
                    def _wait_prev_scatter():
                        pltpu.make_async_copy(
                            rows_v.at[bn], acc.at[didx_ring.at[bn]],
                            ssem.at[bn]).wait()

                    @pl.when(jg < cpw)
                    def _prefetch():
                        pltpu.async_copy(
                            didx_hbm.at[rowbase + jg], didx_ring.at[bn],
                            dsem.at[bn])
                        pltpu.async_copy(
                            hp_hbm.at[sidx_v.at[jnp.minimum(jg, cpw - 1)]],
                            rows_v.at[bn], gsem.at[bn])
                return carry

            lax.fori_loop(0, cpw // NBUF, step_body, 0)
            for i in range(NBUF - LOOKAHEAD):
                j = cpw - (NBUF - LOOKAHEAD) + i
                b = j % NBUF
                pltpu.make_async_copy(
                    rows_v.at[b], acc.at[didx_ring.at[b]], ssem.at[b]).wait()

        with jax.named_scope("mainloop"):
            @pl.when(c == 0)
            def _run0():
                run(CPW_AGG[0], s * CPW_AGG[0])

            @pl.when(c == 1)
            def _run1():
                run(CPW_AGG[1], NS * CPW_AGG[0] + s * CPW_AGG[1])

        with jax.named_scope("barrier2"):
            plsc.subcore_barrier()
        with jax.named_scope("copyout"):
            pltpu.sync_copy(acc.at[pl.ds(r0, ROWS_PT)], out_hbm.at[c, pl.ds(r0, ROWS_PT)])

    return agg_kernel


_deg_kernel = _make_deg_kernel()
_agg64 = _make_agg_kernel(64)
_agg32 = _make_agg_kernel(32)


def _tc1_body(x_ref, w1_ref, d0_ref, d1_ref, o_ref):
    deg = d0_ref[:, 0:1] + d1_ref[:, 0:1] + 1.0
    dis = lax.rsqrt(deg)
    m = jnp.dot(x_ref[...], w1_ref[...], preferred_element_type=jnp.float32)
    o_ref[...] = m * dis


def _tc2_body(p0_ref, p1_ref, hp_ref, d0_ref, d1_ref, b1_ref, w2_ref, o_ref):
    deg = d0_ref[:, 0:1] + d1_ref[:, 0:1] + 1.0
    dis = lax.rsqrt(deg)
    s = p0_ref[...] + p1_ref[...] + hp_ref[...]
    h = jnp.maximum(s * dis + b1_ref[...], 0.0)
    o_ref[...] = jnp.dot(h, w2_ref[...], preferred_element_type=jnp.float32) * dis


def _tc3_body(p0_ref, p1_ref, hp_ref, d0_ref, d1_ref, b2_ref, o_ref):
    deg = d0_ref[:, 0:1] + d1_ref[:, 0:1] + 1.0
    dis = lax.rsqrt(deg)
    o_ref[...] = (p0_ref[...] + p1_ref[...] + hp_ref[...]) * dis + b2_ref[...]


def kernel(x, edge_index, W1, b1, W2, b2):
    src = edge_index[0].astype(jnp.int32)
    dst = edge_index[1].astype(jnp.int32)
    pad = EPAD - E
    src_p = jnp.concatenate(
        [src, jnp.full((pad,), PAD_NODE, jnp.int32)]).reshape(NW * CPW, CH)
    dst_p = jnp.concatenate(
        [dst, jnp.full((pad,), PAD_NODE, jnp.int32)]).reshape(NW * CPW, CH)
    xpad = jnp.zeros((NPAD, x.shape[1]), x.dtype).at[:N].set(x)

    zeros16 = jnp.zeros((NPAD, 16), jnp.float32)
    zeros64 = jnp.zeros((NPAD, 64), jnp.float32)
    zeros32 = jnp.zeros((NPAD, 32), jnp.float32)
    ones = jnp.ones((CH, 16), jnp.float32)

    dparts = _deg_kernel(dst_p, zeros16, ones)
    d0, d1 = dparts[0], dparts[1]

    hp1 = pl.pallas_call(
        _tc1_body,
        out_shape=jax.ShapeDtypeStruct((NPAD, 64), jnp.float32),
    )(xpad, W1, d0, d1)

    p1 = _agg64(hp1, src_p, dst_p, zeros64)

    hp2 = pl.pallas_call(
        _tc2_body,
        out_shape=jax.ShapeDtypeStruct((NPAD, 32), jnp.float32),
    )(p1[0], p1[1], hp1, d0, d1, b1.reshape(1, 64), W2)

    p2 = _agg32(hp2, src_p, dst_p, zeros32)

    z = pl.pallas_call(
        _tc3_body,
        out_shape=jax.ShapeDtypeStruct((NPAD, 32), jnp.float32),
    )(p2[0], p2[1], hp2, d0, d1, b2.reshape(1, 32))

    return z[:N]


# speedup vs baseline: 1.8619x; 1.8619x over previous
import functools

import jax
import jax.numpy as jnp
from jax import lax
from jax.experimental import pallas as pl
from jax.experimental.pallas import tpu as pltpu
from jax.experimental.pallas import tpu_sc as plsc

N = 10000
NPAD = 10240
E = 320000
NC = 2
NS = 16
NW = NC * NS
CH = 128
CPW = 80
EPAD = NW * CPW * CH
ROWS_PT = NPAD // NS
PAD_NODE = N
NBUF = 8
LOOKAHEAD = 4

CPW_AGG = (80, 80)
CPW_DEG = (80, 80)

_MESH = plsc.VectorSubcoreMesh(
    core_axis_name="c", subcore_axis_name="s", num_cores=NC, num_subcores=NS
)


def _make_deg_kernel():
    @functools.partial(
        pl.kernel,
        out_type=jax.ShapeDtypeStruct((NC, NPAD, 16), jnp.float32),
        mesh=_MESH,
        scratch_types=[
            pltpu.VMEM((CPW_DEG[0], CH), jnp.int32),
            pltpu.VMEM((CH, 16), jnp.float32),
            pltpu.VMEM_SHARED((NPAD, 16), jnp.float32),
            pltpu.SemaphoreType.DMA,
        ],
        compiler_params=pltpu.CompilerParams(use_tc_tiling_on_sc=False),
    )
    def deg_kernel(didx_hbm, zeros_hbm, ones_hbm, out_hbm, didx_v, ones_v, acc,
                   sem):
        c = lax.axis_index("c")
        s = lax.axis_index("s")
        r0 = s * ROWS_PT
        pltpu.sync_copy(zeros_hbm.at[pl.ds(r0, ROWS_PT)], acc.at[pl.ds(r0, ROWS_PT)])
        pltpu.sync_copy(ones_hbm, ones_v)

        @pl.when(c == 0)
        def _load0():
            pltpu.sync_copy(didx_hbm.at[pl.ds(s * CPW_DEG[0], CPW_DEG[0])], didx_v)

        @pl.when(c == 1)
        def _load1():
            pltpu.sync_copy(
                didx_hbm.at[pl.ds(NS * CPW_DEG[0] + s * CPW_DEG[1], CPW_DEG[1])],
                didx_v.at[pl.ds(0, CPW_DEG[1])])

        plsc.subcore_barrier()
        cpw = jnp.where(c == 0, CPW_DEG[0], CPW_DEG[1])

        def body(j, carry):
            pltpu.async_copy(ones_v, acc.at[didx_v.at[j]], sem, add=True)
            return carry

        lax.fori_loop(0, cpw, body, 0)

        def drain(j, carry):
            pltpu.make_async_copy(ones_v, acc.at[didx_v.at[j]], sem).wait()
            return carry

        lax.fori_loop(0, cpw, drain, 0)
        plsc.subcore_barrier()
        pltpu.sync_copy(acc.at[pl.ds(r0, ROWS_PT)], out_hbm.at[c, pl.ds(r0, ROWS_PT)])

    return deg_kernel


def _make_agg_kernel(D):
    @functools.partial(
        pl.kernel,
        out_type=jax.ShapeDtypeStruct((NC, NPAD, D), jnp.float32),
        mesh=_MESH,
        scratch_types=[
            pltpu.VMEM((CPW_AGG[0], CH), jnp.int32),
            pltpu.VMEM((NBUF, CH), jnp.int32),
            pltpu.VMEM((NBUF, CH, D), jnp.float32),
            pltpu.VMEM_SHARED((NPAD, D), jnp.float32),
            pltpu.SemaphoreType.DMA((NBUF,)),
            pltpu.SemaphoreType.DMA((NBUF,)),
            pltpu.SemaphoreType.DMA((NBUF,)),
        ],
        compiler_params=pltpu.CompilerParams(use_tc_tiling_on_sc=False),
    )
    def agg_kernel(hp_hbm, sidx_hbm, didx_hbm, zeros_hbm, out_hbm,
                   sidx_v, didx_ring, rows_v, acc, gsem, ssem, dsem):
        c = lax.axis_index("c")
        s = lax.axis_index("s")
        r0 = s * ROWS_PT
        with jax.named_scope("zinit"):
            pltpu.sync_copy(zeros_hbm.at[pl.ds(r0, ROWS_PT)], acc.at[pl.ds(r0, ROWS_PT)])

        with jax.named_scope("idxload"):
            @pl.when(c == 0)
            def _load0():
                pltpu.sync_copy(sidx_hbm.at[pl.ds(s * CPW_AGG[0], CPW_AGG[0])], sidx_v)

            @pl.when(c == 1)
            def _load1():
                base = NS * CPW_AGG[0] + s * CPW_AGG[1]
                pltpu.sync_copy(sidx_hbm.at[pl.ds(base, CPW_AGG[1])],
                                sidx_v.at[pl.ds(0, CPW_AGG[1])])

        with jax.named_scope("barrier1"):
            plsc.subcore_barrier()

        def run(cpw, rowbase):
            for b in range(LOOKAHEAD):
                pltpu.async_copy(didx_hbm.at[rowbase + b], didx_ring.at[b],
                                 dsem.at[b])
                pltpu.async_copy(hp_hbm.at[sidx_v.at[b]], rows_v.at[b], gsem.at[b])

            def step_body(step, carry):
                base = step * NBUF
                for b in range(NBUF):
                    j = base + b
                    pltpu.make_async_copy(
                        hp_hbm.at[sidx_v.at[j]], rows_v.at[b], gsem.at[b]).wait()
                    pltpu.make_async_copy(
                        didx_hbm.at[rowbase + j], didx_ring.at[b],
                        dsem.at[b]).wait()
                    pltpu.async_copy(
                        rows_v.at[b], acc.at[didx_ring.at[b]], ssem.at[b],
                        add=True)
                    bn = (b + LOOKAHEAD) % NBUF
                    jg = j + LOOKAHEAD
                    jp = jg - NBUF

                    @pl.when(jp >= 0)
                    def _wait_prev_scatter():
                        pltpu.make_async_copy(
                            rows_v.at[bn], acc.at[didx_ring.at[bn]],
                            ssem.at[bn]).wait()

                    @pl.when(jg < cpw)
                    def _prefetch():
                        pltpu.async_copy(
                            didx_hbm.at[rowbase + jg], didx_ring.at[bn],
                            dsem.at[bn])
                        pltpu.async_copy(
                            hp_hbm.at[sidx_v.at[jnp.minimum(jg, cpw - 1)]],
                            rows_v.at[bn], gsem.at[bn])
                return carry

            lax.fori_loop(0, cpw // NBUF, step_body, 0)
            for i in range(NBUF - LOOKAHEAD):
                j = cpw - (NBUF - LOOKAHEAD) + i
                b = j % NBUF
                pltpu.make_async_copy(
                    rows_v.at[b], acc.at[didx_ring.at[b]], ssem.at[b]).wait()

        with jax.named_scope("mainloop"):
            @pl.when(c == 0)
            def _run0():
                run(CPW_AGG[0], s * CPW_AGG[0])

            @pl.when(c == 1)
            def _run1():
                run(CPW_AGG[1], NS * CPW_AGG[0] + s * CPW_AGG[1])

        with jax.named_scope("barrier2"):
            plsc.subcore_barrier()
        with jax.named_scope("copyout"):
            pltpu.sync_copy(acc.at[pl.ds(r0, ROWS_PT)], out_hbm.at[c, pl.ds(r0, ROWS_PT)])

    return agg_kernel


_deg_kernel = _make_deg_kernel()
_agg64 = _make_agg_kernel(64)
_agg32 = _make_agg_kernel(32)


def _tc1_body(x_ref, w1_ref, d0_ref, d1_ref, o_ref):
    deg = d0_ref[:, 0:1] + d1_ref[:, 0:1] + 1.0
    dis = lax.rsqrt(deg)
    m = jnp.dot(x_ref[...], w1_ref[...], preferred_element_type=jnp.float32)
    o_ref[...] = m * dis


def _tc2_body(p0_ref, p1_ref, hp_ref, d0_ref, d1_ref, b1_ref, w2_ref, o_ref):
    deg = d0_ref[:, 0:1] + d1_ref[:, 0:1] + 1.0
    dis = lax.rsqrt(deg)
    s = p0_ref[...] + p1_ref[...] + hp_ref[...]
    h = jnp.maximum(s * dis + b1_ref[...], 0.0)
    o_ref[...] = jnp.dot(h, w2_ref[...], preferred_element_type=jnp.float32) * dis


def _tc3_body(p0_ref, p1_ref, hp_ref, d0_ref, d1_ref, b2_ref, o_ref):
    deg = d0_ref[:, 0:1] + d1_ref[:, 0:1] + 1.0
    dis = lax.rsqrt(deg)
    o_ref[...] = (p0_ref[...] + p1_ref[...] + hp_ref[...]) * dis + b2_ref[...]


def kernel(x, edge_index, W1, b1, W2, b2):
    src = edge_index[0].astype(jnp.int32)
    dst = edge_index[1].astype(jnp.int32)
    pad = EPAD - E
    pad_idx = PAD_NODE + (jnp.arange(pad, dtype=jnp.int32) % (NPAD - N))
    src_p = jnp.concatenate([src, pad_idx]).reshape(NW * CPW, CH)
    dst_p = jnp.concatenate([dst, pad_idx]).reshape(NW * CPW, CH)
    xpad = jnp.zeros((NPAD, x.shape[1]), x.dtype).at[:N].set(x)

    zeros16 = jnp.zeros((NPAD, 16), jnp.float32)
    zeros64 = jnp.zeros((NPAD, 64), jnp.float32)
    zeros32 = jnp.zeros((NPAD, 32), jnp.float32)
    ones = jnp.ones((CH, 16), jnp.float32)

    dparts = _deg_kernel(dst_p, zeros16, ones)
    d0, d1 = dparts[0], dparts[1]

    hp1 = pl.pallas_call(
        _tc1_body,
        out_shape=jax.ShapeDtypeStruct((NPAD, 64), jnp.float32),
    )(xpad, W1, d0, d1)

    p1 = _agg64(hp1, src_p, dst_p, zeros64)

    hp2 = pl.pallas_call(
        _tc2_body,
        out_shape=jax.ShapeDtypeStruct((NPAD, 32), jnp.float32),
    )(p1[0], p1[1], hp1, d0, d1, b1.reshape(1, 64), W2)

    p2 = _agg32(hp2, src_p, dst_p, zeros32)

    z = pl.pallas_call(
        _tc3_body,
        out_shape=jax.ShapeDtypeStruct((NPAD, 32), jnp.float32),
    )(p2[0], p2[1], hp2, d0, d1, b2.reshape(1, 32))

    return z[:N]


# speedup vs baseline: 2.0232x; 1.0867x over previous
import functools

import numpy as np
import jax
import jax.numpy as jnp
from jax import lax
from jax.experimental import pallas as pl
from jax.experimental.pallas import tpu as pltpu
from jax.experimental.pallas import tpu_sc as plsc

N = 10000
NPAD = 10240
E = 320000
NC = 2
NS = 16
NW = NC * NS
CH = 128
CPW = 80
NCH = E // CH
REAL_LAST = NCH - (NW - 1) * CPW
PAD_CH = CPW - REAL_LAST
ROWS_PT = NPAD // NS
NBUF = 8
LOOKAHEAD = 4

_PAD_ROWS = N + (np.arange(64 * CH, dtype=np.int32) % (NPAD - N)).reshape(64, CH)

_MESH = plsc.VectorSubcoreMesh(
    core_axis_name="c", subcore_axis_name="s", num_cores=NC, num_subcores=NS
)


def _load_idx(idx_hbm, pad_hbm, idx_v, wid, is_last):
    @pl.when(jnp.logical_not(is_last))
    def _normal():
        pltpu.sync_copy(idx_hbm.at[pl.ds(wid * CPW, CPW)], idx_v)

    @pl.when(is_last)
    def _last():
        pltpu.sync_copy(idx_hbm.at[pl.ds(NCH - REAL_LAST, REAL_LAST)],
                        idx_v.at[pl.ds(0, REAL_LAST)])
        pltpu.sync_copy(pad_hbm.at[pl.ds(0, PAD_CH)],
                        idx_v.at[pl.ds(REAL_LAST, PAD_CH)])


def _make_deg_kernel():
    @functools.partial(
        pl.kernel,
        out_type=jax.ShapeDtypeStruct((NC, NPAD, 16), jnp.float32),
        mesh=_MESH,
        scratch_types=[
            pltpu.VMEM((CPW, CH), jnp.int32),
            pltpu.VMEM((CH, 16), jnp.float32),
            pltpu.VMEM_SHARED((NPAD, 16), jnp.float32),
            pltpu.SemaphoreType.DMA,
        ],
        compiler_params=pltpu.CompilerParams(use_tc_tiling_on_sc=False),
    )
    def deg_kernel(didx_hbm, pad_hbm, zeros_hbm, ones_hbm, out_hbm,
                   didx_v, ones_v, acc, sem):
        c = lax.axis_index("c")
        s = lax.axis_index("s")
        wid = c * NS + s
        is_last = jnp.logical_and(c == NC - 1, s == NS - 1)
        r0 = s * ROWS_PT
        pltpu.sync_copy(zeros_hbm.at[pl.ds(r0, ROWS_PT)], acc.at[pl.ds(r0, ROWS_PT)])
        pltpu.sync_copy(ones_hbm, ones_v)
        _load_idx(didx_hbm, pad_hbm, didx_v, wid, is_last)
        plsc.subcore_barrier()

        def body(j, carry):
            pltpu.async_copy(ones_v, acc.at[didx_v.at[j]], sem, add=True)
            return carry

        lax.fori_loop(0, CPW, body, 0)

        def drain(j, carry):
            pltpu.make_async_copy(ones_v, acc.at[didx_v.at[j]], sem).wait()
            return carry

        lax.fori_loop(0, CPW, drain, 0)
        plsc.subcore_barrier()
        pltpu.sync_copy(acc.at[pl.ds(r0, ROWS_PT)], out_hbm.at[c, pl.ds(r0, ROWS_PT)])

    return deg_kernel


def _make_agg_kernel(D):
    @functools.partial(
        pl.kernel,
        out_type=jax.ShapeDtypeStruct((NC, NPAD, D), jnp.float32),
        mesh=_MESH,
        scratch_types=[
            pltpu.VMEM((CPW, CH), jnp.int32),
            pltpu.VMEM((CPW, CH), jnp.int32),
            pltpu.VMEM((NBUF, CH, D), jnp.float32),
            pltpu.VMEM_SHARED((NPAD, D), jnp.float32),
            pltpu.SemaphoreType.DMA((NBUF,)),
            pltpu.SemaphoreType.DMA((NBUF,)),
        ],
        compiler_params=pltpu.CompilerParams(use_tc_tiling_on_sc=False),
    )
    def agg_kernel(hp_hbm, sidx_hbm, didx_hbm, pad_hbm, zeros_hbm, out_hbm,
                   sidx_v, didx_v, rows_v, acc, gsem, ssem):
        c = lax.axis_index("c")
        s = lax.axis_index("s")
        wid = c * NS + s
        is_last = jnp.logical_and(c == NC - 1, s == NS - 1)
        r0 = s * ROWS_PT
        pltpu.sync_copy(zeros_hbm.at[pl.ds(r0, ROWS_PT)], acc.at[pl.ds(r0, ROWS_PT)])
        _load_idx(sidx_hbm, pad_hbm, sidx_v, wid, is_last)
        _load_idx(didx_hbm, pad_hbm, didx_v, wid, is_last)
        plsc.subcore_barrier()

        for b in range(LOOKAHEAD):
            pltpu.async_copy(hp_hbm.at[sidx_v.at[b]], rows_v.at[b], gsem.at[b])

        def step_body(step, carry):
            base = step * NBUF
            for b in range(NBUF):
                j = base + b
                pltpu.make_async_copy(
                    hp_hbm.at[sidx_v.at[j]], rows_v.at[b], gsem.at[b]).wait()
                pltpu.async_copy(
                    rows_v.at[b], acc.at[didx_v.at[j]], ssem.at[b], add=True)
                bn = (b + LOOKAHEAD) % NBUF
                jg = j + LOOKAHEAD
                jp = jg - NBUF

                @pl.when(jp >= 0)
                def _wait_prev_scatter():
                    pltpu.make_async_copy(
                        rows_v.at[bn],
                        acc.at[didx_v.at[jnp.maximum(jp, 0)]],
                        ssem.at[bn]).wait()

                @pl.when(jg < CPW)
                def _prefetch_gather():
                    pltpu.async_copy(
                        hp_hbm.at[sidx_v.at[jnp.minimum(jg, CPW - 1)]],
                        rows_v.at[bn], gsem.at[bn])
            return carry

        lax.fori_loop(0, CPW // NBUF, step_body, 0)
        for i in range(NBUF - LOOKAHEAD):
            j = CPW - (NBUF - LOOKAHEAD) + i
            b = j % NBUF
            pltpu.make_async_copy(
                rows_v.at[b], acc.at[didx_v.at[j]], ssem.at[b]).wait()
        plsc.subcore_barrier()
        pltpu.sync_copy(acc.at[pl.ds(r0, ROWS_PT)], out_hbm.at[c, pl.ds(r0, ROWS_PT)])

    return agg_kernel


_deg_kernel = _make_deg_kernel()
_agg64 = _make_agg_kernel(64)
_agg32 = _make_agg_kernel(32)


def _dis(dp_ref):
    deg = dp_ref[0, :, 0:1] + dp_ref[1, :, 0:1] + 1.0
    return lax.rsqrt(deg)


def _tc1_body(x_ref, w1_ref, dp_ref, o_ref):
    dis = _dis(dp_ref)
    m = jnp.dot(x_ref[...], w1_ref[...], preferred_element_type=jnp.float32)
    o_ref[0:N, :] = m * dis[0:N]
    o_ref[N:NPAD, :] = jnp.zeros((NPAD - N, o_ref.shape[1]), jnp.float32)


def _tc2_body(p_ref, hp_ref, dp_ref, b1_ref, w2_ref, o_ref):
    dis = _dis(dp_ref)
    sm = p_ref[0] + p_ref[1] + hp_ref[...]
    h = jnp.maximum(sm * dis + b1_ref[...], 0.0)
    o_ref[...] = jnp.dot(h, w2_ref[...], preferred_element_type=jnp.float32) * dis


def _tc3_body(p_ref, hp_ref, dp_ref, b2_ref, o_ref):
    dis = _dis(dp_ref)
    sm = p_ref[0] + p_ref[1] + hp_ref[...]
    o_ref[...] = (sm * dis + b2_ref[...])[0:N]


def kernel(x, edge_index, W1, b1, W2, b2):
    src_c = edge_index[0].astype(jnp.int32).reshape(NCH, CH)
    dst_c = edge_index[1].astype(jnp.int32).reshape(NCH, CH)
    pad_c = jnp.asarray(_PAD_ROWS)

    zeros16 = jnp.zeros((NPAD, 16), jnp.float32)
    zeros64 = jnp.zeros((NPAD, 64), jnp.float32)
    zeros32 = jnp.zeros((NPAD, 32), jnp.float32)
    ones = jnp.ones((CH, 16), jnp.float32)

    dparts = _deg_kernel(dst_c, pad_c, zeros16, ones)

    hp1 = pl.pallas_call(
        _tc1_body,
        out_shape=jax.ShapeDtypeStruct((NPAD, 64), jnp.float32),
    )(x, W1, dparts)

    p1 = _agg64(hp1, src_c, dst_c, pad_c, zeros64)

    hp2 = pl.pallas_call(
        _tc2_body,
        out_shape=jax.ShapeDtypeStruct((NPAD, 32), jnp.float32),
    )(p1, hp1, dparts, b1.reshape(1, 64), W2)

    p2 = _agg32(hp2, src_c, dst_c, pad_c, zeros32)

    z = pl.pallas_call(
        _tc3_body,
        out_shape=jax.ShapeDtypeStruct((N, 32), jnp.float32),
    )(p2, hp2, dparts, b2.reshape(1, 32))

    return z


# speedup vs baseline: 2.1399x; 1.0577x over previous
import functools

import numpy as np
import jax
import jax.numpy as jnp
from jax import lax
from jax.experimental import pallas as pl
from jax.experimental.pallas import tpu as pltpu
from jax.experimental.pallas import tpu_sc as plsc

N = 10000
NPAD = 10240
E = 320000
NC = 2
NS = 16
NW = NC * NS
CH = 128
CPW = 80
NCH = E // CH
REAL_LAST = NCH - (NW - 1) * CPW
PAD_CH = CPW - REAL_LAST
ROWS_PT = NPAD // NS
NBUF = 8
LOOKAHEAD = 4

_PAD_ROWS = N + (np.arange(64 * CH, dtype=np.int32) % (NPAD - N)).reshape(64, CH)

_MESH = plsc.VectorSubcoreMesh(
    core_axis_name="c", subcore_axis_name="s", num_cores=NC, num_subcores=NS
)


def _load_idx(eidx_hbm, row, pad_hbm, idx_v, wid, is_last):
    @pl.when(jnp.logical_not(is_last))
    def _normal():
        pltpu.sync_copy(eidx_hbm.at[row, pl.ds(wid * CPW, CPW)], idx_v)

    @pl.when(is_last)
    def _last():
        pltpu.sync_copy(eidx_hbm.at[row, pl.ds(NCH - REAL_LAST, REAL_LAST)],
                        idx_v.at[pl.ds(0, REAL_LAST)])
        pltpu.sync_copy(pad_hbm.at[pl.ds(0, PAD_CH)],
                        idx_v.at[pl.ds(REAL_LAST, PAD_CH)])


def _make_deg_kernel():
    @functools.partial(
        pl.kernel,
        out_type=jax.ShapeDtypeStruct((NC, NPAD, 16), jnp.float32),
        mesh=_MESH,
        scratch_types=[
            pltpu.VMEM((CPW, CH), jnp.int32),
            pltpu.VMEM((CH, 16), jnp.float32),
            pltpu.VMEM_SHARED((NPAD, 16), jnp.float32),
            pltpu.SemaphoreType.DMA,
        ],
        compiler_params=pltpu.CompilerParams(use_tc_tiling_on_sc=False),
    )
    def deg_kernel(eidx_hbm, pad_hbm, zeros_hbm, ones_hbm, out_hbm,
                   didx_v, ones_v, acc, sem):
        c = lax.axis_index("c")
        s = lax.axis_index("s")
        wid = c * NS + s
        is_last = jnp.logical_and(c == NC - 1, s == NS - 1)
        r0 = s * ROWS_PT
        pltpu.sync_copy(zeros_hbm.at[pl.ds(r0, ROWS_PT)], acc.at[pl.ds(r0, ROWS_PT)])
        pltpu.sync_copy(ones_hbm, ones_v)
        _load_idx(eidx_hbm, 1, pad_hbm, didx_v, wid, is_last)
        plsc.subcore_barrier()

        def body(j, carry):
            pltpu.async_copy(ones_v, acc.at[didx_v.at[j]], sem, add=True)
            return carry

        lax.fori_loop(0, CPW, body, 0)

        def drain(j, carry):
            pltpu.make_async_copy(ones_v, acc.at[didx_v.at[j]], sem).wait()
            return carry

        lax.fori_loop(0, CPW, drain, 0)
        plsc.subcore_barrier()
        pltpu.sync_copy(acc.at[pl.ds(r0, ROWS_PT)], out_hbm.at[c, pl.ds(r0, ROWS_PT)])

    return deg_kernel


def _make_agg_kernel(D):
    @functools.partial(
        pl.kernel,
        out_type=jax.ShapeDtypeStruct((NC, NPAD, D), jnp.float32),
        mesh=_MESH,
        scratch_types=[
            pltpu.VMEM((CPW, CH), jnp.int32),
            pltpu.VMEM((CPW, CH), jnp.int32),
            pltpu.VMEM((NBUF, CH, D), jnp.float32),
            pltpu.VMEM_SHARED((NPAD, D), jnp.float32),
            pltpu.SemaphoreType.DMA((NBUF,)),
            pltpu.SemaphoreType.DMA((NBUF,)),
        ],
        compiler_params=pltpu.CompilerParams(use_tc_tiling_on_sc=False),
    )
    def agg_kernel(hp_hbm, eidx_hbm, pad_hbm, zeros_hbm, out_hbm,
                   sidx_v, didx_v, rows_v, acc, gsem, ssem):
        c = lax.axis_index("c")
        s = lax.axis_index("s")
        wid = c * NS + s
        is_last = jnp.logical_and(c == NC - 1, s == NS - 1)
        r0 = s * ROWS_PT
        pltpu.sync_copy(zeros_hbm.at[pl.ds(r0, ROWS_PT)], acc.at[pl.ds(r0, ROWS_PT)])
        _load_idx(eidx_hbm, 0, pad_hbm, sidx_v, wid, is_last)
        _load_idx(eidx_hbm, 1, pad_hbm, didx_v, wid, is_last)
        plsc.subcore_barrier()

        for b in range(LOOKAHEAD):
            pltpu.async_copy(hp_hbm.at[sidx_v.at[b]], rows_v.at[b], gsem.at[b])

        def step_body(step, carry):
            base = step * NBUF
            for b in range(NBUF):
                j = base + b
                pltpu.make_async_copy(
                    hp_hbm.at[sidx_v.at[j]], rows_v.at[b], gsem.at[b]).wait()
                pltpu.async_copy(
                    rows_v.at[b], acc.at[didx_v.at[j]], ssem.at[b], add=True)
                bn = (b + LOOKAHEAD) % NBUF
                jg = j + LOOKAHEAD
                jp = jg - NBUF

                @pl.when(jp >= 0)
                def _wait_prev_scatter():
                    pltpu.make_async_copy(
                        rows_v.at[bn],
                        acc.at[didx_v.at[jnp.maximum(jp, 0)]],
                        ssem.at[bn]).wait()

                @pl.when(jg < CPW)
                def _prefetch_gather():
                    pltpu.async_copy(
                        hp_hbm.at[sidx_v.at[jnp.minimum(jg, CPW - 1)]],
                        rows_v.at[bn], gsem.at[bn])
            return carry

        lax.fori_loop(0, CPW // NBUF, step_body, 0)
        for i in range(NBUF - LOOKAHEAD):
            j = CPW - (NBUF - LOOKAHEAD) + i
            b = j % NBUF
            pltpu.make_async_copy(
                rows_v.at[b], acc.at[didx_v.at[j]], ssem.at[b]).wait()
        plsc.subcore_barrier()
        pltpu.sync_copy(acc.at[pl.ds(r0, ROWS_PT)], out_hbm.at[c, pl.ds(r0, ROWS_PT)])

    return agg_kernel


_deg_kernel = _make_deg_kernel()
_agg64 = _make_agg_kernel(64)
_agg32 = _make_agg_kernel(32)


def _dis(dp_ref):
    deg = dp_ref[0, :, 0:1] + dp_ref[1, :, 0:1] + 1.0
    return lax.rsqrt(deg)


def _tc1_body(x_ref, w1_ref, dp_ref, o_ref):
    dis = _dis(dp_ref)
    m = jnp.dot(x_ref[...], w1_ref[...], preferred_element_type=jnp.float32)
    o_ref[0:N, :] = m * dis[0:N]
    o_ref[N:NPAD, :] = jnp.zeros((NPAD - N, o_ref.shape[1]), jnp.float32)


def _tc2_body(p_ref, hp_ref, dp_ref, b1_ref, w2_ref, o_ref):
    dis = _dis(dp_ref)
    sm = p_ref[0] + p_ref[1] + hp_ref[...]
    h = jnp.maximum(sm * dis + b1_ref[...], 0.0)
    o_ref[...] = jnp.dot(h, w2_ref[...], preferred_element_type=jnp.float32) * dis


def _tc3_body(p_ref, hp_ref, dp_ref, b2_ref, o_ref):
    dis = _dis(dp_ref)
    sm = p_ref[0] + p_ref[1] + hp_ref[...]
    o_ref[...] = (sm * dis + b2_ref[...])[0:N]


def kernel(x, edge_index, W1, b1, W2, b2):
    eidx = edge_index.astype(jnp.int32).reshape(2, NCH, CH)
    pad_c = jnp.asarray(_PAD_ROWS)

    zeros16 = jnp.zeros((NPAD, 16), jnp.float32)
    zeros64 = jnp.zeros((NPAD, 64), jnp.float32)
    zeros32 = jnp.zeros((NPAD, 32), jnp.float32)
    ones = jnp.ones((CH, 16), jnp.float32)

    dparts = _deg_kernel(eidx, pad_c, zeros16, ones)

    hp1 = pl.pallas_call(
        _tc1_body,
        out_shape=jax.ShapeDtypeStruct((NPAD, 64), jnp.float32),
    )(x, W1, dparts)

    p1 = _agg64(hp1, eidx, pad_c, zeros64)

    hp2 = pl.pallas_call(
        _tc2_body,
        out_shape=jax.ShapeDtypeStruct((NPAD, 32), jnp.float32),
    )(p1, hp1, dparts, b1.reshape(1, 64), W2)

    p2 = _agg32(hp2, eidx, pad_c, zeros32)

    z = pl.pallas_call(
        _tc3_body,
        out_shape=jax.ShapeDtypeStruct((N, 32), jnp.float32),
    )(p2, hp2, dparts, b2.reshape(1, 32))

    return z
